# baseline (speedup 1.0000x reference)
import numpy as np
import jax
import jax.numpy as jnp
from jax import lax
from jax.experimental import pallas as pl
from jax.experimental.pallas import tpu as pltpu
from jax.experimental.pallas import tpu_sc as plsc

D = 64
SEQ = 200
BATCH = 4096
B = BATCH * SEQ
NC, NS = 2, 16
NW = NC * NS
BPW = B // NW
CB = 2
C = CB * SEQ
NCHUNK = BPW // C
IDXM = 100
NSUB = C // IDXM
KPB = SEQ // IDXM
IROWS = BPW // IDXM
SCALE = 8.0
NBUF = 3


def _pos_pattern() -> jax.Array:
    position = np.arange(SEQ)[:, np.newaxis]
    k = np.arange(D)[np.newaxis, :]
    i = k // 2
    angle_rates = 1 / np.power(10000, 2 * i / np.float32(D))
    angle_rads = position * angle_rates
    angle_rads[:, 0::2] = np.sin(angle_rads[:, 0::2])
    angle_rads[:, 1::2] = np.cos(angle_rads[:, 1::2])
    return jnp.asarray(angle_rads.astype(np.float32))


def _make_mesh():
    return plsc.VectorSubcoreMesh(
        core_axis_name="c", subcore_axis_name="s",
        num_cores=NC, num_subcores=NS)


def _emb_body(x_hbm, table_hbm, pos_hbm, out_hbm,
              idx_v, rows0, rows1, rows2, pos_v,
              sg0, sg1, sg2, ss0, ss1, ss2):
    wid = lax.axis_index("s") * NC + lax.axis_index("c")
    bbase = wid * (BPW // SEQ)
    rows = (rows0, rows1, rows2)
    sg = (sg0, sg1, sg2)
    ss = (ss0, ss1, ss2)

    pltpu.sync_copy(pos_hbm, pos_v)
    pltpu.sync_copy(x_hbm.at[pl.ds(wid * IROWS, IROWS)], idx_v)

    def fire_gathers(g, buf, sem):
        for j in range(NSUB):
            pltpu.make_async_copy(
                table_hbm.at[idx_v.at[g * NSUB + j]],
                buf.at[j // KPB, pl.ds((j % KPB) * IDXM, IDXM)],
                sem).start()

    def drain_gathers(g, buf, sem):
        for j in range(NSUB):
            pltpu.make_async_copy(
                table_hbm.at[idx_v.at[g * NSUB + j]],
                buf.at[j // KPB, pl.ds((j % KPB) * IDXM, IDXM)],
                sem).wait()

    def fire_scatter(g, buf, sem):
        pltpu.make_async_copy(
            buf, out_hbm.at[pl.ds(bbase + g * CB, CB)], sem).start()

    def wait_scatter(g, buf, sem):
        pltpu.make_async_copy(
            buf, out_hbm.at[pl.ds(bbase + g * CB, CB)], sem).wait()

    def fma(buf):
        def fma_body(r, _):
            for j in range(D // 16):
                sl = pl.ds(j * 16, 16)
                p = pos_v[r, sl]
                for q in range(CB):
                    buf[q, r, sl] = buf[q, r, sl] * SCALE + p
            return ()
        lax.fori_loop(0, SEQ, fma_body, (), unroll=4)

    def step(g, b):
        nb = (b + 1) % NBUF

        @pl.when(g >= NBUF - 1)
        def _wait_prev_scatter():
            wait_scatter(g - (NBUF - 1), rows[nb], ss[nb])

        @pl.when(g + 1 < NCHUNK)
        def _fire_next_gather():
            fire_gathers(g + 1, rows[nb], sg[nb])

        drain_gathers(g, rows[b], sg[b])
        fma(rows[b])
        fire_scatter(g, rows[b], ss[b])

    fire_gathers(0, rows0, sg0)

    def outer(i, _):
        g0 = i * NBUF
        for b in range(NBUF):
            step(g0 + b, b)
        return ()
    nfull = (NCHUNK // NBUF) * NBUF
    lax.fori_loop(0, NCHUNK // NBUF, outer, ())

    for g in range(nfull, NCHUNK):
        step(g, g % NBUF)

    for g in range(NCHUNK - (NBUF - 1), NCHUNK):
        wait_scatter(g, rows[g % NBUF], ss[g % NBUF])


VW = 384
NVW = 2604
WPT = 82
WR = VW // 2
VTAIL = 1000000 - NVW * VW
GRP = 6


def _tr_body(tt_hbm, scr_hbm, blk0, blk1, blk2, out0, out1,
             sr0, sr1, sr2, sw0, sw1):
    wid = lax.axis_index("s") * NC + lax.axis_index("c")
    blk = (blk0, blk1, blk2)
    out = (out0, out1)
    sr = (sr0, sr1, sr2)
    sw = (sw0, sw1)
    w0 = wid * WPT
    wlim = jnp.minimum(w0 + WPT, NVW)

    iota = lax.broadcasted_iota(jnp.int32, (16,), 0)
    rowbase = lax.shift_right_logical(iota, 1)
    colpar = (iota & 1) * D

    def rd(w, b):
        return pltpu.make_async_copy(
            tt_hbm.at[:, pl.ds(pl.multiple_of(w * VW, VW), VW)],
            blk[b], sr[b])

    def wr(w, b):
        return pltpu.make_async_copy(
            out[b], scr_hbm.at[pl.ds(pl.multiple_of(w * WR, WR), WR)], sw[b])

    def transpose(bb, ob):
        @plsc.parallel_loop(0, D, unroll=4)
        def f_body(f):
            col_v = colpar + f
            for cb in range(VW // 16):
                row_v = rowbase + 8 * cb
                x = blk[bb][f, pl.ds(cb * 16, 16)]
                plsc.store_scatter(out[ob], [row_v, col_v], x)

    rd(w0, 0).start()
    rd(w0 + 1, 1).start()

    def outer(i6, _):
        for k in range(GRP):
            w = w0 + i6 * GRP + k
            bb, ob = k % 3, k % 2

            @pl.when(w < wlim)
            def _do():
                @pl.when(w + 2 < wlim)
                def _prefetch():
                    rd(w + 2, (k + 2) % 3).start()
                rd(w, bb).wait()

                @pl.when(w >= w0 + 2)
                def _drain_prev_write():
                    wr(w - 2, ob).wait()
                transpose(bb, ob)
                wr(w, ob).start()
        return ()
    lax.fori_loop(0, (WPT + GRP - 1) // GRP, outer, ())

    wr(wlim - 2, 0).wait()
    wr(wlim - 1, 1).wait()

def _transpose_table(table):
    tt = table.T
    return pl.kernel(
        _tr_body,
        out_type=jax.ShapeDtypeStruct((500000, 2 * D), jnp.float32),
        mesh=_make_mesh(),
        scratch_types=[
            pltpu.VMEM((D, VW), jnp.float32),
            pltpu.VMEM((D, VW), jnp.float32),
            pltpu.VMEM((D, VW), jnp.float32),
            pltpu.VMEM((WR, 2 * D), jnp.float32),
            pltpu.VMEM((WR, 2 * D), jnp.float32),
            pltpu.SemaphoreType.DMA,
            pltpu.SemaphoreType.DMA,
            pltpu.SemaphoreType.DMA,
            pltpu.SemaphoreType.DMA,
            pltpu.SemaphoreType.DMA,
        ],
        compiler_params=pltpu.CompilerParams(
            use_tc_tiling_on_sc=True, needs_layout_passes=False),
    )(tt)


@jax.jit
def kernel(x, table):
    x_flat = x.reshape(B // IDXM, IDXM)
    pos_pat = _pos_pattern()
    t_scr = _transpose_table(table)
    upd = table[NVW * VW:].reshape(VTAIL // 2, 2 * D)
    t_scr = lax.dynamic_update_slice(t_scr, upd, (NVW * VW // 2, 0))
    t_lin = t_scr.reshape(1000000, D)
    return pl.kernel(
        _emb_body,
        out_type=jax.ShapeDtypeStruct((BATCH, SEQ, D), jnp.float32),
        mesh=_make_mesh(),
        scratch_types=[
            pltpu.VMEM((IROWS, IDXM), jnp.int32),
            pltpu.VMEM((CB, SEQ, D), jnp.float32),
            pltpu.VMEM((CB, SEQ, D), jnp.float32),
            pltpu.VMEM((CB, SEQ, D), jnp.float32),
            pltpu.VMEM((SEQ, D), jnp.float32),
            pltpu.SemaphoreType.DMA,
            pltpu.SemaphoreType.DMA,
            pltpu.SemaphoreType.DMA,
            pltpu.SemaphoreType.DMA,
            pltpu.SemaphoreType.DMA,
            pltpu.SemaphoreType.DMA,
        ],
        compiler_params=pltpu.CompilerParams(use_tc_tiling_on_sc=False),
    )(x_flat, t_lin, pos_pat)


# speedup vs baseline: 1.2031x; 1.2031x over previous
import numpy as np
import jax
import jax.numpy as jnp
from jax import lax
from jax.experimental import pallas as pl
from jax.experimental.pallas import tpu as pltpu
from jax.experimental.pallas import tpu_sc as plsc

D = 64
SEQ = 200
BATCH = 4096
B = BATCH * SEQ
NC, NS = 2, 16
NW = NC * NS
BPW = B // NW
CB = 2
C = CB * SEQ
NCHUNK = BPW // C
IDXM = 100
NSUB = C // IDXM
KPB = SEQ // IDXM
IROWS = BPW // IDXM
SCALE = 8.0
NBUF = 3


def _pos_pattern() -> jax.Array:
    position = np.arange(SEQ)[:, np.newaxis]
    k = np.arange(D)[np.newaxis, :]
    i = k // 2
    angle_rates = 1 / np.power(10000, 2 * i / np.float32(D))
    angle_rads = position * angle_rates
    angle_rads[:, 0::2] = np.sin(angle_rads[:, 0::2])
    angle_rads[:, 1::2] = np.cos(angle_rads[:, 1::2])
    return jnp.asarray(angle_rads.astype(np.float32))


def _make_mesh():
    return plsc.VectorSubcoreMesh(
        core_axis_name="c", subcore_axis_name="s",
        num_cores=NC, num_subcores=NS)


def _emb_body(x_hbm, table_hbm, pos_hbm, out_hbm,
              idx_v, rows0, rows1, rows2, pos_v,
              sg0, sg1, sg2, ss0, ss1, ss2):
    wid = lax.axis_index("s") * NC + lax.axis_index("c")
    bbase = wid * (BPW // SEQ)
    rows = (rows0, rows1, rows2)
    sg = (sg0, sg1, sg2)
    ss = (ss0, ss1, ss2)

    pltpu.sync_copy(pos_hbm, pos_v)
    pltpu.sync_copy(x_hbm.at[pl.ds(wid * IROWS, IROWS)], idx_v)

    def fire_gathers(g, buf, sem):
        for j in range(NSUB):
            pltpu.make_async_copy(
                table_hbm.at[idx_v.at[g * NSUB + j]],
                buf.at[j // KPB, pl.ds((j % KPB) * IDXM, IDXM)],
                sem).start()

    def drain_gathers(g, buf, sem):
        for j in range(NSUB):
            pltpu.make_async_copy(
                table_hbm.at[idx_v.at[g * NSUB + j]],
                buf.at[j // KPB, pl.ds((j % KPB) * IDXM, IDXM)],
                sem).wait()

    def fire_scatter(g, buf, sem):
        pltpu.make_async_copy(
            buf, out_hbm.at[pl.ds(bbase + g * CB, CB)], sem).start()

    def wait_scatter(g, buf, sem):
        pltpu.make_async_copy(
            buf, out_hbm.at[pl.ds(bbase + g * CB, CB)], sem).wait()

    def fma(buf):
        def fma_body(r, _):
            for j in range(D // 16):
                sl = pl.ds(j * 16, 16)
                p = pos_v[r, sl]
                for q in range(CB):
                    buf[q, r, sl] = buf[q, r, sl] * SCALE + p
            return ()
        lax.fori_loop(0, SEQ, fma_body, (), unroll=4)

    def step(g, b):
        nb = (b + 1) % NBUF

        @pl.when(g >= NBUF - 1)
        def _wait_prev_scatter():
            wait_scatter(g - (NBUF - 1), rows[nb], ss[nb])

        @pl.when(g + 1 < NCHUNK)
        def _fire_next_gather():
            fire_gathers(g + 1, rows[nb], sg[nb])

        drain_gathers(g, rows[b], sg[b])
        fma(rows[b])
        fire_scatter(g, rows[b], ss[b])

    fire_gathers(0, rows0, sg0)

    def outer(i, _):
        g0 = i * NBUF
        for b in range(NBUF):
            step(g0 + b, b)
        return ()
    nfull = (NCHUNK // NBUF) * NBUF
    lax.fori_loop(0, NCHUNK // NBUF, outer, ())

    for g in range(nfull, NCHUNK):
        step(g, g % NBUF)

    for g in range(NCHUNK - (NBUF - 1), NCHUNK):
        wait_scatter(g, rows[g % NBUF], ss[g % NBUF])


@jax.jit
def kernel(x, table):
    x_flat = x.reshape(B // IDXM, IDXM)
    pos_pat = _pos_pattern()
    return pl.kernel(
        _emb_body,
        out_type=jax.ShapeDtypeStruct((BATCH, SEQ, D), jnp.float32),
        mesh=_make_mesh(),
        scratch_types=[
            pltpu.VMEM((IROWS, IDXM), jnp.int32),
            pltpu.VMEM((CB, SEQ, D), jnp.float32),
            pltpu.VMEM((CB, SEQ, D), jnp.float32),
            pltpu.VMEM((CB, SEQ, D), jnp.float32),
            pltpu.VMEM((SEQ, D), jnp.float32),
            pltpu.SemaphoreType.DMA,
            pltpu.SemaphoreType.DMA,
            pltpu.SemaphoreType.DMA,
            pltpu.SemaphoreType.DMA,
            pltpu.SemaphoreType.DMA,
            pltpu.SemaphoreType.DMA,
        ],
        compiler_params=pltpu.CompilerParams(use_tc_tiling_on_sc=False),
    )(x_flat, table, pos_pat)
